# baseline (device time: 10556 ns/iter reference)
import os

import jax
import jax.numpy as jnp
from jax import lax
from jax.experimental import pallas as pl
from jax.experimental.pallas import tpu as pltpu

N_DEV = 8
N_GLOBAL = 8192
EPS = 1e-5
CHUNKS = 4
NO_COMM = os.environ.get("NO_COMM") == "1"


def kernel(x, gamma):
    m, n_per = x.shape
    m_c = m // CHUNKS
    sub = m_c // 128
    g2 = gamma.reshape(1, n_per)

    def body(x_hbm, g_ref, out_hbm, x_vmem, out_vmem, *scratch):
        comms = scratch[:CHUNKS]
        send_sems = scratch[CHUNKS : 2 * CHUNKS]
        recv_sems = scratch[2 * CHUNKS : 3 * CHUNKS]
        in_sems = scratch[3 * CHUNKS]
        out_sems = scratch[3 * CHUNKS + 1]
        my = lax.axis_index("i")

        if not NO_COMM:
            barrier_sem = pltpu.get_barrier_semaphore()
            for p in range(N_DEV):

                @pl.when(p != my)
                def _():
                    pl.semaphore_signal(
                        barrier_sem,
                        inc=1,
                        device_id=(p,),
                        device_id_type=pl.DeviceIdType.MESH,
                    )

        in_copies = []
        for c in range(CHUNKS):
            cp = pltpu.make_async_copy(
                x_hbm.at[pl.ds(c * m_c, m_c), :],
                x_vmem.at[pl.ds(c * m_c, m_c), :],
                in_sems.at[c],
            )
            cp.start()
            in_copies.append(cp)

        ones_v = jnp.ones((1, n_per), jnp.float32)

        for c in range(CHUNKS):
            in_copies[c].wait()
            xc = x_vmem[pl.ds(c * m_c, m_c), :]
            sq = xc * xc
            part = lax.dot_general(
                ones_v,
                sq,
                (((1,), (1,)), ((), ())),
                preferred_element_type=jnp.float32,
            )
            packed = part.reshape(sub, 128)

            for p in range(N_DEV):

                @pl.when(p == my)
                def _():
                    comms[c][p] = packed

            if not NO_COMM:
                if c == 0:
                    pl.semaphore_wait(barrier_sem, N_DEV - 1)
                for p in range(N_DEV):

                    @pl.when(p != my)
                    def _():
                        rdma = pltpu.make_async_remote_copy(
                            src_ref=comms[c].at[my],
                            dst_ref=comms[c].at[my],
                            send_sem=send_sems[c].at[p],
                            recv_sem=recv_sems[c].at[my],
                            device_id=(p,),
                            device_id_type=pl.DeviceIdType.MESH,
                        )
                        rdma.start()

        out_copies = []
        for c in range(CHUNKS):
            if not NO_COMM:
                for p in range(N_DEV):

                    @pl.when(p != my)
                    def _():
                        recv = pltpu.make_async_remote_copy(
                            src_ref=comms[c].at[p],
                            dst_ref=comms[c].at[p],
                            send_sem=send_sems[c].at[p],
                            recv_sem=recv_sems[c].at[p],
                            device_id=(p,),
                            device_id_type=pl.DeviceIdType.MESH,
                        )
                        recv.wait_recv()

            total = jnp.sum(comms[c][...], axis=0)
            inv = lax.rsqrt(total / N_GLOBAL + EPS)
            for i in range(sub):
                inv_blk = inv[i : i + 1, :].reshape(128, 1)
                r0 = c * m_c + i * 128
                xb = x_vmem[pl.ds(r0, 128), :]
                out_vmem[pl.ds(r0, 128), :] = g_ref[...] * (xb * inv_blk)

            cp = pltpu.make_async_copy(
                out_vmem.at[pl.ds(c * m_c, m_c), :],
                out_hbm.at[pl.ds(c * m_c, m_c), :],
                out_sems.at[c],
            )
            cp.start()
            out_copies.append(cp)

        for cp in out_copies:
            cp.wait()
        for c in range(CHUNKS if not NO_COMM else 0):
            for p in range(N_DEV):

                @pl.when(p != my)
                def _():
                    send = pltpu.make_async_remote_copy(
                        src_ref=comms[c].at[my],
                        dst_ref=comms[c].at[my],
                        send_sem=send_sems[c].at[p],
                        recv_sem=recv_sems[c].at[my],
                        device_id=(p,),
                        device_id_type=pl.DeviceIdType.MESH,
                    )
                    send.wait_send()

    return pl.pallas_call(
        body,
        out_shape=jax.ShapeDtypeStruct((m, n_per), jnp.float32),
        in_specs=[
            pl.BlockSpec(memory_space=pl.ANY),
            pl.BlockSpec(memory_space=pltpu.VMEM),
        ],
        out_specs=pl.BlockSpec(memory_space=pl.ANY),
        scratch_shapes=(
            [pltpu.VMEM((m, n_per), jnp.float32)]
            + [pltpu.VMEM((m, n_per), jnp.float32)]
            + [pltpu.VMEM((N_DEV, m // CHUNKS // 128, 128), jnp.float32)]
            * CHUNKS
            + [pltpu.SemaphoreType.DMA((N_DEV,))] * CHUNKS
            + [pltpu.SemaphoreType.DMA((N_DEV,))] * CHUNKS
            + [pltpu.SemaphoreType.DMA((CHUNKS,))]
            + [pltpu.SemaphoreType.DMA((CHUNKS,))]
        ),
        compiler_params=pltpu.CompilerParams(
            collective_id=None if NO_COMM else 0
        ),
    )(x, g2)
